# baseline (speedup 1.0000x reference)
import functools

import jax
import jax.numpy as jnp
from jax import lax
from jax.experimental import pallas as pl
from jax.experimental.pallas import tpu as pltpu
from jax.experimental.pallas import tpu_sc as plsc

NC = 2
NS = 16
NG = 256
INV_STD = 1.0 / (1.0 + 1e-5) ** 0.5


def _mm_bias_relu_body(x_ref, w_ref, b_ref, o_ref):
    acc = jnp.dot(x_ref[...], w_ref[...], preferred_element_type=jnp.float32)
    o_ref[...] = jnp.maximum(acc + b_ref[...], 0.0)


def _mm_bias_relu(x, w, b, nb):
    n, k = x.shape
    m = w.shape[1]
    return pl.pallas_call(
        _mm_bias_relu_body,
        grid=(n // nb,),
        in_specs=[pl.BlockSpec((nb, k), lambda i: (i, 0)),
                  pl.BlockSpec((k, m), lambda i: (0, 0)),
                  pl.BlockSpec((1, m), lambda i: (0, 0))],
        out_specs=pl.BlockSpec((nb, m), lambda i: (i, 0)),
        out_shape=jax.ShapeDtypeStruct((n, m), jnp.float32),
    )(x, w, b.reshape(1, m))


def _mm_bias_body(x_ref, w_ref, b_ref, o_ref):
    o_ref[...] = (jnp.dot(x_ref[...], w_ref[...],
                          preferred_element_type=jnp.float32) + b_ref[...])


def _edge_term(e1, w1e, b1, eb):
    e, k = e1.shape
    m = w1e.shape[1]
    return pl.pallas_call(
        _mm_bias_body,
        grid=(e // eb,),
        in_specs=[pl.BlockSpec((eb, k), lambda i: (i, 0)),
                  pl.BlockSpec((k, m), lambda i: (0, 0)),
                  pl.BlockSpec((1, m), lambda i: (0, 0))],
        out_specs=pl.BlockSpec((eb, m), lambda i: (i, 0)),
        out_shape=jax.ShapeDtypeStruct((e, m), jnp.float32),
    )(e1, w1e, b1.reshape(1, m))


def _ab_body(h_ref, wd_ref, ws_ref, a_ref, b_ref):
    h = h_ref[...]
    a_ref[...] = jnp.dot(h, wd_ref[...], preferred_element_type=jnp.float32)
    b_ref[...] = jnp.dot(h, ws_ref[...], preferred_element_type=jnp.float32)


def _node_terms(h, w1d, w1s, nb):
    n, din = h.shape
    m = w1d.shape[1]
    spec_o = pl.BlockSpec((nb, m), lambda i: (i, 0))
    spec_w = pl.BlockSpec((din, m), lambda i: (0, 0))
    return pl.pallas_call(
        _ab_body,
        grid=(n // nb,),
        in_specs=[pl.BlockSpec((nb, din), lambda i: (i, 0)), spec_w, spec_w],
        out_specs=[spec_o, spec_o],
        out_shape=[jax.ShapeDtypeStruct((n, m), jnp.float32),
                   jax.ShapeDtypeStruct((n, m), jnp.float32)],
    )(h, w1d, w1s)


def _make_post_body(nv):
    def body(*refs):
        s_refs = refs[:2 * nv]
        w2_ref, d0_ref, d1_ref, b2_ref, g_ref, b_ref, o_ref = refs[2 * nv:]
        acc = jnp.dot(s_refs[0][...], w2_ref[0],
                      preferred_element_type=jnp.float32)
        for q in range(1, 2 * nv):
            acc = acc + jnp.dot(s_refs[q][...], w2_ref[q // 2],
                                preferred_element_type=jnp.float32)
        deg = d0_ref[...] + d1_ref[...]
        denom = jnp.maximum(deg, 1.0)
        mask = (deg > 0.0).astype(jnp.float32)
        agg = acc / denom + b2_ref[...] * mask
        o_ref[...] = jnp.maximum(agg * INV_STD * g_ref[...] + b_ref[...], 0.0)
    return body


def _post(s_list, w2, d0, d1, b2, g, b, nb):
    nv = len(s_list)
    npad = s_list[0].shape[0] // 2
    k2 = s_list[0].shape[1]
    dout = w2.shape[1]
    s_specs = []
    for _ in range(nv):
        s_specs.append(pl.BlockSpec((nb, k2), lambda i: (i, 0)))
        s_specs.append(
            pl.BlockSpec((nb, k2), lambda i, _n=npad // nb: (_n + i, 0)))
    s_args = [s for s in s_list for _ in range(2)]
    return pl.pallas_call(
        _make_post_body(nv),
        grid=(npad // nb,),
        in_specs=s_specs + [
            pl.BlockSpec((nv, k2, dout), lambda i: (0, 0, 0)),
            pl.BlockSpec((nb, 1), lambda i: (i, 0)),
            pl.BlockSpec((nb, 1), lambda i: (i, 0)),
            pl.BlockSpec((1, dout), lambda i: (0, 0)),
            pl.BlockSpec((1, dout), lambda i: (0, 0)),
            pl.BlockSpec((1, dout), lambda i: (0, 0))],
        out_specs=pl.BlockSpec((nb, dout), lambda i: (i, 0)),
        out_shape=jax.ShapeDtypeStruct((npad, dout), jnp.float32),
    )(*s_args, w2.reshape(nv, k2, dout),
      d0, d1, b2.reshape(1, dout), g.reshape(1, dout), b.reshape(1, dout))


def _pool_body(batch_ref, h_ref, wfc_ref, bfc_ref, o_ref, acc_ref, cnt_ref):
    i = pl.program_id(0)

    @pl.when(i == 0)
    def _init():
        acc_ref[...] = jnp.zeros_like(acc_ref)
        cnt_ref[...] = jnp.zeros_like(cnt_ref)

    bids = batch_ref[0]
    onehot = (bids == lax.broadcasted_iota(jnp.int32, (NG, 1), 0))
    onehot = onehot.astype(jnp.float32)
    acc_ref[...] += jnp.dot(onehot, h_ref[...],
                            preferred_element_type=jnp.float32)
    cnt_ref[...] += jnp.sum(onehot, axis=1, keepdims=True)

    @pl.when(i == pl.num_programs(0) - 1)
    def _fin():
        pooled = acc_ref[...] / jnp.maximum(cnt_ref[...], 1.0)
        logits = jnp.dot(pooled, wfc_ref[...],
                         preferred_element_type=jnp.float32) + bfc_ref[...]
        o_ref[...] = jax.nn.sigmoid(logits)


def _pool_head(h, batch, wfc, bfc, nb):
    n, dh = h.shape
    nout = wfc.shape[1]
    batch3 = batch.reshape(n // nb, 1, nb)
    return pl.pallas_call(
        _pool_body,
        grid=(n // nb,),
        in_specs=[pl.BlockSpec((1, 1, nb), lambda i: (i, 0, 0)),
                  pl.BlockSpec((nb, dh), lambda i: (i, 0)),
                  pl.BlockSpec((dh, nout), lambda i: (0, 0)),
                  pl.BlockSpec((1, nout), lambda i: (0, 0))],
        out_specs=pl.BlockSpec((NG, nout), lambda i: (0, 0)),
        out_shape=jax.ShapeDtypeStruct((NG, nout), jnp.float32),
        scratch_shapes=[pltpu.VMEM((NG, dh), jnp.float32),
                        pltpu.VMEM((NG, 1), jnp.float32)],
    )(batch3, h, wfc, bfc.reshape(1, nout))


def _sc_mesh():
    return plsc.VectorSubcoreMesh(core_axis_name="c", subcore_axis_name="s",
                                  num_cores=NC, num_subcores=NS)


_CH = 80


def _deg_kernel(npad, e):
    ept = e // (NC * NS)
    nchunks = ept // _CH
    rpt = npad // NS

    @functools.partial(
        pl.kernel,
        out_type=jax.ShapeDtypeStruct((NC * npad,), jnp.float32),
        mesh=_sc_mesh(),
        scratch_types=[
            pltpu.VMEM((_CH,), jnp.int32),
            pltpu.VMEM((_CH,), jnp.float32),
            pltpu.VMEM((rpt,), jnp.float32),
            pltpu.VMEM_SHARED((npad,), jnp.float32),
        ],
    )
    def deg_k(dst_hbm, out_hbm, idx_v, ones_v, z_v, deg_sp):
        cid = lax.axis_index("c")
        sid = lax.axis_index("s")

        for j in range(_CH // 16):
            ones_v[pl.ds(j * 16, 16)] = jnp.ones((16,), jnp.float32)

        def _z(i, c):
            z_v[pl.ds(i * 16, 16)] = jnp.zeros((16,), jnp.float32)
            return c
        lax.fori_loop(0, rpt // 16, _z, 0)
        pltpu.sync_copy(z_v, deg_sp.at[pl.ds(sid * rpt, rpt)])
        plsc.subcore_barrier()

        base0 = (cid * NS + sid) * ept

        def _chunk(k, c):
            pltpu.sync_copy(dst_hbm.at[pl.ds(base0 + k * _CH, _CH)], idx_v)
            pltpu.sync_copy(ones_v, deg_sp.at[idx_v], add=True)
            return c
        lax.fori_loop(0, nchunks, _chunk, 0)

        plsc.subcore_barrier()
        pltpu.sync_copy(deg_sp.at[pl.ds(sid * rpt, rpt)], z_v)
        pltpu.sync_copy(z_v, out_hbm.at[pl.ds(cid * npad + sid * rpt, rpt)])

    return deg_k


def _edge_kernel(npad, e, k2):
    ch = _CH // 2
    ept = e // (NC * NS)
    nchunks = ept // ch
    rpt = npad // NS
    assert nchunks % 2 == 0 and nchunks >= 4

    @functools.partial(
        pl.kernel,
        out_type=jax.ShapeDtypeStruct((2 * npad, k2), jnp.float32),
        mesh=_sc_mesh(),
        compiler_params=pltpu.CompilerParams(use_tc_tiling_on_sc=False),
        scratch_types=[
            pltpu.VMEM((ch, k2), jnp.float32),
            pltpu.VMEM((ch, k2), jnp.float32),
            pltpu.VMEM((ch, k2), jnp.float32),
            pltpu.VMEM((ch, k2), jnp.float32),
            pltpu.VMEM((ch, k2), jnp.float32),
            pltpu.VMEM((ch, k2), jnp.float32),
            [pltpu.VMEM((ch,), jnp.int32)] * 8,
            pltpu.VMEM_SHARED((npad, k2), jnp.float32),
            pltpu.SemaphoreType.DMA,
            pltpu.SemaphoreType.DMA,
            pltpu.SemaphoreType.DMA,
            pltpu.SemaphoreType.DMA,
            pltpu.SemaphoreType.DMA,
            [pltpu.SemaphoreType.DMA] * 4,
        ],
    )
    def edge_k(a_hbm, b_hbm, c_hbm, src_hbm, dst_hbm, zeros_hbm, out_hbm,
               av_0, bv_0, cv_0, av_1, bv_1, cv_1, idx_ring, s_sp,
               sem_a, sem_b, sem_c, sem_s0, sem_s1, sem_i):
        cid = lax.axis_index("c")
        sid = lax.axis_index("s")

        pltpu.sync_copy(zeros_hbm.at[pl.ds(sid * rpt, rpt)],
                        s_sp.at[pl.ds(sid * rpt, rpt)])
        plsc.subcore_barrier()

        base0 = (cid * NS + sid) * ept
        data = ((av_0, bv_0, cv_0), (av_1, bv_1, cv_1))
        idx = tuple((idx_ring[2 * j], idx_ring[2 * j + 1]) for j in range(4))
        sem_s = (sem_s0, sem_s1)

        def issue_idx(k, j):
            pltpu.async_copy(dst_hbm.at[pl.ds(base0 + k * ch, ch)],
                             idx[j][0], sem_i[j])
            pltpu.async_copy(src_hbm.at[pl.ds(base0 + k * ch, ch)],
                             idx[j][1], sem_i[j])

        def wait_idx(j):
            pltpu.make_async_copy(dst_hbm.at[pl.ds(base0, ch)],
                                  idx[j][0], sem_i[j]).wait()
            pltpu.make_async_copy(src_hbm.at[pl.ds(base0, ch)],
                                  idx[j][1], sem_i[j]).wait()

        def issue_gathers(k, d, j):
            av, bv, cv = data[d]
            pltpu.async_copy(a_hbm.at[idx[j][0]], av, sem_a)
            pltpu.async_copy(b_hbm.at[idx[j][1]], bv, sem_b)
            pltpu.async_copy(c_hbm.at[pl.ds(base0 + k * ch, ch)],
                             cv, sem_c)

        def wait_gathers(d):
            av, bv, cv = data[d]
            pltpu.make_async_copy(c_hbm.at[pl.ds(base0, ch)], av, sem_a).wait()
            pltpu.make_async_copy(c_hbm.at[pl.ds(base0, ch)], bv, sem_b).wait()
            pltpu.make_async_copy(c_hbm.at[pl.ds(base0, ch)], cv, sem_c).wait()

        def compute(d):
            av, bv, cv = data[d]

            @plsc.parallel_loop(0, ch, step=1, unroll=4)
            def _row(i):
                for j in range(k2 // 16):
                    sl = pl.ds(j * 16, 16)
                    av[i, sl] = jnp.maximum(
                        av[i, sl] + bv[i, sl] + cv[i, sl], 0.0)

        def issue_scatter(d, j):
            pltpu.async_copy(data[d][0], s_sp.at[idx[j][0]],
                             sem_s[d], add=True)

        def wait_scatter(d):
            pltpu.make_async_copy(data[d][0], s_sp.at[idx[d][0]],
                                  sem_s[d]).wait()

        def steady(k, k2m, k4m):
            wait_scatter((k2m + 1) % 2)
            issue_idx(k + 3, (k4m + 3) % 4)
            wait_idx((k4m + 1) % 4)
            issue_gathers(k + 1, (k2m + 1) % 2, (k4m + 1) % 4)
            wait_gathers(k2m)
            compute(k2m)
            issue_scatter(k2m, k4m)

        for t in range(4):
            issue_idx(t, t)
        wait_idx(0)
        issue_gathers(0, 0, 0)
        wait_idx(1)
        issue_gathers(1, 1, 1)
        wait_gathers(0)
        compute(0)
        issue_scatter(0, 0)
        steady(1, 1, 1)

        def _quad(qo, c):
            k = 4 * qo + 2
            steady(k, 0, 2)
            steady(k + 1, 1, 3)
            steady(k + 2, 0, 0)
            steady(k + 3, 1, 1)
            return c
        lax.fori_loop(0, (nchunks - 6) // 4, _quad, 0)

        n0 = nchunks - 4
        steady(n0, 0, 2)
        wait_scatter(0)
        wait_idx(0)
        issue_gathers(n0 + 2, 0, 0)
        wait_gathers(1)
        compute(1)
        issue_scatter(1, 3)
        wait_scatter(1)
        wait_idx(1)
        issue_gathers(n0 + 3, 1, 1)
        wait_gathers(0)
        compute(0)
        issue_scatter(0, 0)
        wait_scatter(0)
        wait_gathers(1)
        compute(1)
        issue_scatter(1, 1)
        wait_scatter(1)

        plsc.subcore_barrier()
        pltpu.sync_copy(s_sp.at[pl.ds(sid * rpt, rpt)],
                        out_hbm.at[pl.ds(cid * npad + sid * rpt, rpt)])

    return edge_k


def kernel(x, edge_index, edge_attr, batch, W_edge, b_edge, W_node, b_node,
           c0_W1, c0_b1, c0_W2, c0_b2, bn0_g, bn0_b,
           c1_W1, c1_b1, c1_W2, c1_b2, bn1_g, bn1_b,
           c2_W1, c2_b1, c2_W2, c2_b2, bn2_g, bn2_b, W_fc, b_fc):
    n, d_feat = x.shape
    e = edge_index.shape[1]
    src = edge_index[0]
    dst = edge_index[1]
    npad = ((n + 639) // 640) * 640

    xp = jnp.pad(x, ((0, npad - n), (0, 0)))
    batchp = jnp.pad(batch, (0, npad - n), constant_values=NG)

    degp = _deg_kernel(npad, e)(dst)
    d0 = degp[:npad].reshape(npad, 1)
    d1 = degp[npad:].reshape(npad, 1)

    h = _mm_bias_relu(xp, W_node, b_node, nb=2048)
    e1 = _mm_bias_relu(edge_attr, W_edge, b_edge, nb=8000)

    layers = [(c0_W1, c0_b1, c0_W2, c0_b2, bn0_g, bn0_b),
              (c1_W1, c1_b1, c1_W2, c1_b2, bn1_g, bn1_b),
              (c2_W1, c2_b1, c2_W2, c2_b2, bn2_g, bn2_b)]

    ccs = []
    for (w1, b1, *_rest) in layers:
        din = 128
        for iv in range(w1.shape[1] // 128):
            cs = slice(iv * 128, (iv + 1) * 128)
            ccs.append(_edge_term(e1, w1[2 * din:, cs], b1[cs], eb=8000))
    zeros = jnp.zeros((npad, 128), jnp.float32)

    ci = 0
    for (w1, b1, w2, b2, g, b) in layers:
        din = h.shape[1]
        ktot = w1.shape[1]
        nv = ktot // 128
        s_list = []
        for iv in range(nv):
            cs = slice(iv * 128, (iv + 1) * 128)
            a, bb = _node_terms(h, w1[:din, cs], w1[din:2 * din, cs], nb=2048)
            s_list.append(_edge_kernel(npad, e, 128)(
                a, bb, ccs[ci], src, dst, zeros))
            ci += 1
        h = _post(s_list, w2, d0, d1, b2, g, b, nb=2048)

    return _pool_head(h, batchp, W_fc, b_fc, nb=2048)


# speedup vs baseline: 1.0110x; 1.0110x over previous
import functools

import jax
import jax.numpy as jnp
from jax import lax
from jax.experimental import pallas as pl
from jax.experimental.pallas import tpu as pltpu
from jax.experimental.pallas import tpu_sc as plsc

NC = 2
NS = 16
NG = 256
INV_STD = 1.0 / (1.0 + 1e-5) ** 0.5


def _mm_bias_relu_body(x_ref, w_ref, b_ref, o_ref):
    acc = jnp.dot(x_ref[...], w_ref[...], preferred_element_type=jnp.float32)
    o_ref[...] = jnp.maximum(acc + b_ref[...], 0.0)


def _mm_bias_relu(x, w, b, nb):
    n, k = x.shape
    m = w.shape[1]
    return pl.pallas_call(
        _mm_bias_relu_body,
        grid=(n // nb,),
        in_specs=[pl.BlockSpec((nb, k), lambda i: (i, 0)),
                  pl.BlockSpec((k, m), lambda i: (0, 0)),
                  pl.BlockSpec((1, m), lambda i: (0, 0))],
        out_specs=pl.BlockSpec((nb, m), lambda i: (i, 0)),
        out_shape=jax.ShapeDtypeStruct((n, m), jnp.float32),
    )(x, w, b.reshape(1, m))


def _mm_bias_body(x_ref, w_ref, b_ref, o_ref):
    o_ref[...] = (jnp.dot(x_ref[...], w_ref[...],
                          preferred_element_type=jnp.float32) + b_ref[...])


def _edge_term(e1, w1e, b1, eb):
    e, k = e1.shape
    m = w1e.shape[1]
    return pl.pallas_call(
        _mm_bias_body,
        grid=(e // eb,),
        in_specs=[pl.BlockSpec((eb, k), lambda i: (i, 0)),
                  pl.BlockSpec((k, m), lambda i: (0, 0)),
                  pl.BlockSpec((1, m), lambda i: (0, 0))],
        out_specs=pl.BlockSpec((eb, m), lambda i: (i, 0)),
        out_shape=jax.ShapeDtypeStruct((e, m), jnp.float32),
    )(e1, w1e, b1.reshape(1, m))


def _ab_body(h_ref, wd_ref, ws_ref, a_ref, b_ref):
    h = h_ref[...]
    a_ref[...] = jnp.dot(h, wd_ref[...], preferred_element_type=jnp.float32)
    b_ref[...] = jnp.dot(h, ws_ref[...], preferred_element_type=jnp.float32)


def _node_terms(h, w1d, w1s, nb):
    n, din = h.shape
    m = w1d.shape[1]
    spec_o = pl.BlockSpec((nb, m), lambda i: (i, 0))
    spec_w = pl.BlockSpec((din, m), lambda i: (0, 0))
    return pl.pallas_call(
        _ab_body,
        grid=(n // nb,),
        in_specs=[pl.BlockSpec((nb, din), lambda i: (i, 0)), spec_w, spec_w],
        out_specs=[spec_o, spec_o],
        out_shape=[jax.ShapeDtypeStruct((n, m), jnp.float32),
                   jax.ShapeDtypeStruct((n, m), jnp.float32)],
    )(h, w1d, w1s)


def _make_post_body(nv):
    def body(*refs):
        s_refs = refs[:2 * nv]
        w2_ref, d0_ref, d1_ref, b2_ref, g_ref, b_ref, o_ref = refs[2 * nv:]
        acc = jnp.dot(s_refs[0][...], w2_ref[0],
                      preferred_element_type=jnp.float32)
        for q in range(1, 2 * nv):
            acc = acc + jnp.dot(s_refs[q][...], w2_ref[q // 2],
                                preferred_element_type=jnp.float32)
        deg = d0_ref[...] + d1_ref[...]
        denom = jnp.maximum(deg, 1.0)
        mask = (deg > 0.0).astype(jnp.float32)
        agg = acc / denom + b2_ref[...] * mask
        o_ref[...] = jnp.maximum(agg * INV_STD * g_ref[...] + b_ref[...], 0.0)
    return body


def _post(s_list, w2, d0, d1, b2, g, b, nb):
    nv = len(s_list)
    npad = s_list[0].shape[0] // 2
    k2 = s_list[0].shape[1]
    dout = w2.shape[1]
    s_specs = []
    for _ in range(nv):
        s_specs.append(pl.BlockSpec((nb, k2), lambda i: (i, 0)))
        s_specs.append(
            pl.BlockSpec((nb, k2), lambda i, _n=npad // nb: (_n + i, 0)))
    s_args = [s for s in s_list for _ in range(2)]
    return pl.pallas_call(
        _make_post_body(nv),
        grid=(npad // nb,),
        in_specs=s_specs + [
            pl.BlockSpec((nv, k2, dout), lambda i: (0, 0, 0)),
            pl.BlockSpec((nb, 1), lambda i: (i, 0)),
            pl.BlockSpec((nb, 1), lambda i: (i, 0)),
            pl.BlockSpec((1, dout), lambda i: (0, 0)),
            pl.BlockSpec((1, dout), lambda i: (0, 0)),
            pl.BlockSpec((1, dout), lambda i: (0, 0))],
        out_specs=pl.BlockSpec((nb, dout), lambda i: (i, 0)),
        out_shape=jax.ShapeDtypeStruct((npad, dout), jnp.float32),
    )(*s_args, w2.reshape(nv, k2, dout),
      d0, d1, b2.reshape(1, dout), g.reshape(1, dout), b.reshape(1, dout))


def _pool_body(batch_ref, h_ref, wfc_ref, bfc_ref, o_ref, acc_ref, cnt_ref):
    i = pl.program_id(0)

    @pl.when(i == 0)
    def _init():
        acc_ref[...] = jnp.zeros_like(acc_ref)
        cnt_ref[...] = jnp.zeros_like(cnt_ref)

    bids = batch_ref[0]
    onehot = (bids == lax.broadcasted_iota(jnp.int32, (NG, 1), 0))
    onehot = onehot.astype(jnp.float32)
    acc_ref[...] += jnp.dot(onehot, h_ref[...],
                            preferred_element_type=jnp.float32)
    cnt_ref[...] += jnp.sum(onehot, axis=1, keepdims=True)

    @pl.when(i == pl.num_programs(0) - 1)
    def _fin():
        pooled = acc_ref[...] / jnp.maximum(cnt_ref[...], 1.0)
        logits = jnp.dot(pooled, wfc_ref[...],
                         preferred_element_type=jnp.float32) + bfc_ref[...]
        o_ref[...] = jax.nn.sigmoid(logits)


def _pool_head(h, batch, wfc, bfc, nb):
    n, dh = h.shape
    nout = wfc.shape[1]
    batch3 = batch.reshape(n // nb, 1, nb)
    return pl.pallas_call(
        _pool_body,
        grid=(n // nb,),
        in_specs=[pl.BlockSpec((1, 1, nb), lambda i: (i, 0, 0)),
                  pl.BlockSpec((nb, dh), lambda i: (i, 0)),
                  pl.BlockSpec((dh, nout), lambda i: (0, 0)),
                  pl.BlockSpec((1, nout), lambda i: (0, 0))],
        out_specs=pl.BlockSpec((NG, nout), lambda i: (0, 0)),
        out_shape=jax.ShapeDtypeStruct((NG, nout), jnp.float32),
        scratch_shapes=[pltpu.VMEM((NG, dh), jnp.float32),
                        pltpu.VMEM((NG, 1), jnp.float32)],
    )(batch3, h, wfc, bfc.reshape(1, nout))


def _sc_mesh():
    return plsc.VectorSubcoreMesh(core_axis_name="c", subcore_axis_name="s",
                                  num_cores=NC, num_subcores=NS)


_CH = 80


def _deg_kernel(npad, e):
    ept = e // (NC * NS)
    nchunks = ept // _CH
    rpt = npad // NS

    @functools.partial(
        pl.kernel,
        out_type=jax.ShapeDtypeStruct((NC * npad,), jnp.float32),
        mesh=_sc_mesh(),
        scratch_types=[
            pltpu.VMEM((_CH,), jnp.int32),
            pltpu.VMEM((_CH,), jnp.float32),
            pltpu.VMEM((rpt,), jnp.float32),
            pltpu.VMEM_SHARED((npad,), jnp.float32),
        ],
    )
    def deg_k(dst_hbm, out_hbm, idx_v, ones_v, z_v, deg_sp):
        cid = lax.axis_index("c")
        sid = lax.axis_index("s")

        for j in range(_CH // 16):
            ones_v[pl.ds(j * 16, 16)] = jnp.ones((16,), jnp.float32)

        def _z(i, c):
            z_v[pl.ds(i * 16, 16)] = jnp.zeros((16,), jnp.float32)
            return c
        lax.fori_loop(0, rpt // 16, _z, 0)
        pltpu.sync_copy(z_v, deg_sp.at[pl.ds(sid * rpt, rpt)])
        plsc.subcore_barrier()

        base0 = (cid * NS + sid) * ept

        def _chunk(k, c):
            pltpu.sync_copy(dst_hbm.at[pl.ds(base0 + k * _CH, _CH)], idx_v)
            pltpu.sync_copy(ones_v, deg_sp.at[idx_v], add=True)
            return c
        lax.fori_loop(0, nchunks, _chunk, 0)

        plsc.subcore_barrier()
        pltpu.sync_copy(deg_sp.at[pl.ds(sid * rpt, rpt)], z_v)
        pltpu.sync_copy(z_v, out_hbm.at[pl.ds(cid * npad + sid * rpt, rpt)])

    return deg_k


def _edge_kernel(npad, e, k2):
    ch = _CH // 2
    ept = e // (NC * NS)
    nchunks = ept // ch
    rpt = npad // NS
    assert nchunks % 2 == 0 and nchunks >= 4

    @functools.partial(
        pl.kernel,
        out_type=jax.ShapeDtypeStruct((2 * npad, k2), jnp.float32),
        mesh=_sc_mesh(),
        compiler_params=pltpu.CompilerParams(use_tc_tiling_on_sc=False),
        scratch_types=[
            pltpu.VMEM((ch, k2), jnp.float32),
            pltpu.VMEM((ch, k2), jnp.float32),
            pltpu.VMEM((ch, k2), jnp.float32),
            pltpu.VMEM((ch, k2), jnp.float32),
            pltpu.VMEM((ch, k2), jnp.float32),
            pltpu.VMEM((ch, k2), jnp.float32),
            [pltpu.VMEM((ch,), jnp.int32)] * 8,
            pltpu.VMEM_SHARED((npad, k2), jnp.float32),
            pltpu.SemaphoreType.DMA,
            pltpu.SemaphoreType.DMA,
            pltpu.SemaphoreType.DMA,
            pltpu.SemaphoreType.DMA,
            pltpu.SemaphoreType.DMA,
            [pltpu.SemaphoreType.DMA] * 4,
        ],
    )
    def edge_k(a_hbm, b_hbm, c_hbm, src_hbm, dst_hbm, zeros_hbm, out_hbm,
               av_0, bv_0, cv_0, av_1, bv_1, cv_1, idx_ring, s_sp,
               sem_a, sem_b, sem_c, sem_s0, sem_s1, sem_i):
        cid = lax.axis_index("c")
        sid = lax.axis_index("s")

        pltpu.sync_copy(zeros_hbm.at[pl.ds(sid * rpt, rpt)],
                        s_sp.at[pl.ds(sid * rpt, rpt)])
        plsc.subcore_barrier()

        base0 = (cid * NS + sid) * ept
        data = ((av_0, bv_0, cv_0), (av_1, bv_1, cv_1))
        idx = tuple((idx_ring[2 * j], idx_ring[2 * j + 1]) for j in range(4))
        sem_s = (sem_s0, sem_s1)

        def issue_idx(k, j):
            pltpu.async_copy(dst_hbm.at[pl.ds(base0 + k * ch, ch)],
                             idx[j][0], sem_i[j])
            pltpu.async_copy(src_hbm.at[pl.ds(base0 + k * ch, ch)],
                             idx[j][1], sem_i[j])

        def wait_idx(j):
            pltpu.make_async_copy(dst_hbm.at[pl.ds(base0, ch)],
                                  idx[j][0], sem_i[j]).wait()
            pltpu.make_async_copy(src_hbm.at[pl.ds(base0, ch)],
                                  idx[j][1], sem_i[j]).wait()

        def issue_gathers(k, d, j):
            av, bv, cv = data[d]
            pltpu.async_copy(a_hbm.at[idx[j][0]], av, sem_a)
            pltpu.async_copy(b_hbm.at[idx[j][1]], bv, sem_b)
            pltpu.async_copy(c_hbm.at[pl.ds(base0 + k * ch, ch)],
                             cv, sem_c)

        def wait_gathers(d):
            av, bv, cv = data[d]
            pltpu.make_async_copy(c_hbm.at[pl.ds(base0, ch)], av, sem_a).wait()
            pltpu.make_async_copy(c_hbm.at[pl.ds(base0, ch)], bv, sem_b).wait()
            pltpu.make_async_copy(c_hbm.at[pl.ds(base0, ch)], cv, sem_c).wait()

        def compute(d):
            av, bv, cv = data[d]

            def _row(i, c2):
                for j in range(k2 // 16):
                    sl = pl.ds(j * 16, 16)
                    av[i, sl] = jnp.maximum(
                        av[i, sl] + bv[i, sl] + cv[i, sl], 0.0)
                return c2
            lax.fori_loop(0, ch, _row, 0)

        def issue_scatter(d, j):
            pltpu.async_copy(data[d][0], s_sp.at[idx[j][0]],
                             sem_s[d], add=True)

        def wait_scatter(d):
            pltpu.make_async_copy(data[d][0], s_sp.at[idx[d][0]],
                                  sem_s[d]).wait()

        def steady(k, k2m, k4m):
            wait_scatter((k2m + 1) % 2)
            issue_idx(k + 3, (k4m + 3) % 4)
            wait_idx((k4m + 1) % 4)
            issue_gathers(k + 1, (k2m + 1) % 2, (k4m + 1) % 4)
            wait_gathers(k2m)
            compute(k2m)
            issue_scatter(k2m, k4m)

        for t in range(4):
            issue_idx(t, t)
        wait_idx(0)
        issue_gathers(0, 0, 0)
        wait_idx(1)
        issue_gathers(1, 1, 1)
        wait_gathers(0)
        compute(0)
        issue_scatter(0, 0)
        steady(1, 1, 1)

        def _quad(qo, c):
            k = 4 * qo + 2
            steady(k, 0, 2)
            steady(k + 1, 1, 3)
            steady(k + 2, 0, 0)
            steady(k + 3, 1, 1)
            return c
        lax.fori_loop(0, (nchunks - 6) // 4, _quad, 0)

        n0 = nchunks - 4
        steady(n0, 0, 2)
        wait_scatter(0)
        wait_idx(0)
        issue_gathers(n0 + 2, 0, 0)
        wait_gathers(1)
        compute(1)
        issue_scatter(1, 3)
        wait_scatter(1)
        wait_idx(1)
        issue_gathers(n0 + 3, 1, 1)
        wait_gathers(0)
        compute(0)
        issue_scatter(0, 0)
        wait_scatter(0)
        wait_gathers(1)
        compute(1)
        issue_scatter(1, 1)
        wait_scatter(1)

        plsc.subcore_barrier()
        pltpu.sync_copy(s_sp.at[pl.ds(sid * rpt, rpt)],
                        out_hbm.at[pl.ds(cid * npad + sid * rpt, rpt)])

    return edge_k


def kernel(x, edge_index, edge_attr, batch, W_edge, b_edge, W_node, b_node,
           c0_W1, c0_b1, c0_W2, c0_b2, bn0_g, bn0_b,
           c1_W1, c1_b1, c1_W2, c1_b2, bn1_g, bn1_b,
           c2_W1, c2_b1, c2_W2, c2_b2, bn2_g, bn2_b, W_fc, b_fc):
    n, d_feat = x.shape
    e = edge_index.shape[1]
    src = edge_index[0]
    dst = edge_index[1]
    npad = ((n + 639) // 640) * 640

    xp = jnp.pad(x, ((0, npad - n), (0, 0)))
    batchp = jnp.pad(batch, (0, npad - n), constant_values=NG)

    degp = _deg_kernel(npad, e)(dst)
    d0 = degp[:npad].reshape(npad, 1)
    d1 = degp[npad:].reshape(npad, 1)

    h = _mm_bias_relu(xp, W_node, b_node, nb=2048)
    e1 = _mm_bias_relu(edge_attr, W_edge, b_edge, nb=8000)

    layers = [(c0_W1, c0_b1, c0_W2, c0_b2, bn0_g, bn0_b),
              (c1_W1, c1_b1, c1_W2, c1_b2, bn1_g, bn1_b),
              (c2_W1, c2_b1, c2_W2, c2_b2, bn2_g, bn2_b)]

    ccs = []
    for (w1, b1, *_rest) in layers:
        din = 128
        for iv in range(w1.shape[1] // 128):
            cs = slice(iv * 128, (iv + 1) * 128)
            ccs.append(_edge_term(e1, w1[2 * din:, cs], b1[cs], eb=8000))
    zeros = jnp.zeros((npad, 128), jnp.float32)

    ci = 0
    for (w1, b1, w2, b2, g, b) in layers:
        din = h.shape[1]
        ktot = w1.shape[1]
        nv = ktot // 128
        s_list = []
        for iv in range(nv):
            cs = slice(iv * 128, (iv + 1) * 128)
            a, bb = _node_terms(h, w1[:din, cs], w1[din:2 * din, cs], nb=2048)
            s_list.append(_edge_kernel(npad, e, 128)(
                a, bb, ccs[ci], src, dst, zeros))
            ci += 1
        h = _post(s_list, w2, d0, d1, b2, g, b, nb=2048)

    return _pool_head(h, batchp, W_fc, b_fc, nb=2048)
